# baseline (speedup 1.0000x reference)
import jax
import jax.numpy as jnp
from jax.experimental import pallas as pl
from jax.experimental.pallas import tpu as pltpu


def kernel(input_ids, positional_encoding_table):
    seq_len = input_ids.shape[1]
    model_dim = positional_encoding_table.shape[1]

    def body(table_ref, out_ref, sem):
        copy = pltpu.make_async_copy(
            table_ref.at[pl.ds(0, seq_len), :], out_ref, sem
        )
        copy.start()
        copy.wait()

    return pl.pallas_call(
        body,
        out_shape=jax.ShapeDtypeStruct((seq_len, model_dim),
                                       positional_encoding_table.dtype),
        in_specs=[pl.BlockSpec(memory_space=pl.ANY)],
        out_specs=pl.BlockSpec(memory_space=pl.ANY),
        scratch_shapes=[pltpu.SemaphoreType.DMA],
    )(positional_encoding_table)
